# baseline (speedup 1.0000x reference)
import functools
import jax
import jax.numpy as jnp
from jax import lax
from jax.experimental import pallas as pl
from jax.experimental.pallas import tpu as pltpu
from jax.experimental.pallas import tpu_sc as plsc

N = 10000
NP = 10240
E = 320000
NC, NS, LANES = 2, 16, 16
NW = NC * NS
EPW = E // NW
ZUNROLL = 16
EUNROLL = 5


def _zero_vmem(ref, nwords):
    zeros = jnp.zeros((LANES,), jnp.float32)
    per_it = ZUNROLL * LANES
    assert nwords % per_it == 0

    def zbody(i, _):
        for j in range(ZUNROLL):
            ref[pl.ds(i * per_it + j * LANES, LANES)] = zeros
        return 0

    lax.fori_loop(0, nwords // per_it, zbody, 0)


RUNROLL = 5


def _staged_reduce(shared, red2, acc_v, out_row, sid, nwords, nchunks):
    cw = nwords // nchunks
    nw = cw // NS
    assert nw % (LANES * RUNROLL) == 0 and nw % 128 == 0
    assert red2.shape == (NS, nw)
    for c in range(nchunks):
        if c > 0:
            plsc.subcore_barrier()
        pltpu.sync_copy(acc_v.at[pl.ds(c * cw, cw)], shared.at[sid])
        plsc.subcore_barrier()
        start = sid * nw
        pltpu.sync_copy(shared.at[:, pl.ds(start, nw)], red2)

        def abody(i, _):
            for j in range(RUNROLL):
                o = (i * RUNROLL + j) * LANES
                ds = pl.ds(o, LANES)
                s01 = red2[0, ds] + red2[1, ds]
                s23 = red2[2, ds] + red2[3, ds]
                s45 = red2[4, ds] + red2[5, ds]
                s67 = red2[6, ds] + red2[7, ds]
                s89 = red2[8, ds] + red2[9, ds]
                sab = red2[10, ds] + red2[11, ds]
                scd = red2[12, ds] + red2[13, ds]
                sef = red2[14, ds] + red2[15, ds]
                red2[0, ds] = (((s01 + s23) + (s45 + s67))
                               + ((s89 + sab) + (scd + sef)))
            return 0

        lax.fori_loop(0, nw // (LANES * RUNROLL), abody, 0)
        pltpu.sync_copy(red2.at[0],
                        out_row.at[pl.ds(c * cw + start, nw)])


def _deg_body(dst_hbm, out_hbm, dst_v, acc_v, red2, shared):
    cid = lax.axis_index("c")
    sid = lax.axis_index("s")
    wid = sid * NC + cid
    _zero_vmem(acc_v, NP)
    pltpu.sync_copy(dst_hbm.at[pl.ds(wid * EPW, EPW)], dst_v)

    ones = jnp.ones((LANES,), jnp.float32)

    def ebody(i, _):
        for j in range(EUNROLL):
            d16 = dst_v[pl.ds((i * EUNROLL + j) * LANES, LANES)]
            plsc.addupdate_scatter(acc_v, [d16], ones)
        return 0

    lax.fori_loop(0, EPW // (LANES * EUNROLL), ebody, 0)
    _staged_reduce(shared, red2, acc_v, out_hbm.at[cid], sid, NP, 1)


@functools.lru_cache(maxsize=None)
def _deg_kernel():
    return functools.partial(
        pl.kernel,
        out_type=jax.ShapeDtypeStruct((NC, NP), jnp.float32),
        mesh=plsc.VectorSubcoreMesh(core_axis_name="c", subcore_axis_name="s"),
        compiler_params=pltpu.CompilerParams(needs_layout_passes=False),
        scratch_types=[
            pltpu.VMEM((EPW,), jnp.int32),
            pltpu.VMEM((NP,), jnp.float32),
            pltpu.VMEM((NS, NP // NS), jnp.float32),
            pltpu.VMEM_SHARED((NS, NP), jnp.float32),
        ],
    )(_deg_body)


def _make_agg(F):
    EBLK = 2000
    NBLK = EPW // EBLK
    NGRP = 2 if F >= 6 else 1
    FG = F // NGRP

    def body(g_hbm, src_hbm, dst_hbm, out_hbm, g_v, acc_v, src_v, dst_v,
             red2, shared):
        cid = lax.axis_index("c")
        sid = lax.axis_index("s")
        base = (sid * NC + cid) * EPW
        _zero_vmem(acc_v, NP * F)

        for grp in range(NGRP):
            pltpu.sync_copy(g_hbm.at[pl.ds(grp * FG * NP, FG * NP)], g_v)

            def ebody(i, _):
                for j in range(EUNROLL):
                    off = (i * EUNROLL + j) * LANES
                    s16 = src_v[pl.ds(off, LANES)]
                    d16 = dst_v[pl.ds(off, LANES)]
                    for f in range(FG):
                        v = plsc.load_gather(g_v, [s16 + f * NP])
                        plsc.addupdate_scatter(
                            acc_v, [d16 + (grp * FG + f) * NP], v)
                return 0

            for b in range(NBLK):
                pltpu.sync_copy(src_hbm.at[pl.ds(base + b * EBLK, EBLK)],
                                src_v)
                pltpu.sync_copy(dst_hbm.at[pl.ds(base + b * EBLK, EBLK)],
                                dst_v)
                lax.fori_loop(0, EBLK // (LANES * EUNROLL), ebody, 0)

        _staged_reduce(shared, red2, acc_v, out_hbm.at[cid], sid, NP * F, F)

    return functools.partial(
        pl.kernel,
        out_type=jax.ShapeDtypeStruct((NC, NP * F), jnp.float32),
        mesh=plsc.VectorSubcoreMesh(core_axis_name="c", subcore_axis_name="s"),
        compiler_params=pltpu.CompilerParams(needs_layout_passes=False),
        scratch_types=[
            pltpu.VMEM((NP * FG,), jnp.float32),
            pltpu.VMEM((NP * F,), jnp.float32),
            pltpu.VMEM((EBLK,), jnp.int32),
            pltpu.VMEM((EBLK,), jnp.int32),
            pltpu.VMEM((NS, NP // NS), jnp.float32),
            pltpu.VMEM_SHARED((NS, NP), jnp.float32),
        ],
    )(body)


_make_agg = functools.lru_cache(maxsize=None)(_make_agg)


def _sum2(ref):
    return ref[0] + ref[1]


def _head_body(x_ref, w1t_ref, degp_ref, dinv_ref, g1_ref):
    deg = _sum2(degp_ref) + 1.0
    dinv = lax.rsqrt(deg)
    dinv_ref[...] = dinv
    h1t = lax.dot_general(w1t_ref[...], x_ref[...],
                          (((1,), (1,)), ((), ())),
                          preferred_element_type=jnp.float32)
    h1p = jnp.concatenate([h1t, jnp.zeros((5, NP - N), jnp.float32)], axis=1)
    g1_ref[...] = dinv * h1p


def _tc_head(x, w1t, degp):
    return pl.pallas_call(
        _head_body,
        out_shape=[
            jax.ShapeDtypeStruct((NP,), jnp.float32),
            jax.ShapeDtypeStruct((5, NP), jnp.float32),
        ],
    )(x, w1t, degp)


def _layer1_body(sp_ref, dinv_ref, g_ref, b_ref, x_ref, gn_ref):
    dinv = dinv_ref[...]
    u = dinv * (_sum2(sp_ref) + g_ref[...])
    xv = u + b_ref[...]
    x_ref[...] = xv
    gn_ref[...] = dinv * xv


def _tc_layer1(sp, dinv, g, bt):
    F = g.shape[0]
    return pl.pallas_call(
        _layer1_body,
        out_shape=[
            jax.ShapeDtypeStruct((F, NP), jnp.float32),
            jax.ShapeDtypeStruct((F, NP), jnp.float32),
        ],
    )(sp, dinv, g, bt)


def _layer2_body(sp_ref, dinv_ref, g_ref, wt_ref, b_ref, x_ref, gn_ref):
    dinv = dinv_ref[...]
    u = dinv * (_sum2(sp_ref) + g_ref[...])
    xv = jnp.dot(wt_ref[...], u, preferred_element_type=jnp.float32) + b_ref[...]
    x_ref[...] = xv
    gn_ref[...] = dinv * xv


def _tc_layer2(sp, dinv, g, wt, bt):
    FO = wt.shape[0]
    return pl.pallas_call(
        _layer2_body,
        out_shape=[
            jax.ShapeDtypeStruct((FO, NP), jnp.float32),
            jax.ShapeDtypeStruct((FO, NP), jnp.float32),
        ],
    )(sp, dinv, g, wt, bt)


def _layer3_body(sp_ref, dinv_ref, g_ref, wt_ref, b_ref, wlt_ref, bl_ref,
                 x_ref, o_ref):
    dinv = dinv_ref[...]
    u = dinv * (_sum2(sp_ref) + g_ref[...])
    xv = jnp.dot(wt_ref[...], u, preferred_element_type=jnp.float32) + b_ref[...]
    x_ref[...] = xv
    o_ref[...] = jnp.dot(wlt_ref[...], xv,
                         preferred_element_type=jnp.float32) + bl_ref[...]


def _tc_layer3(sp, dinv, g, wt, bt, wlt, blt):
    FO = wt.shape[0]
    FL = wlt.shape[0]
    return pl.pallas_call(
        _layer3_body,
        out_shape=[
            jax.ShapeDtypeStruct((FO, NP), jnp.float32),
            jax.ShapeDtypeStruct((FL, NP), jnp.float32),
        ],
    )(sp, dinv, g, wt, bt, wlt, blt)


def kernel(x, edge_index, W1, b1, W2, b2, W3, b3, Wl, bl):
    src = edge_index[0].astype(jnp.int32)
    dst = edge_index[1].astype(jnp.int32)

    degp = _deg_kernel()(dst)
    dinv, g1 = _tc_head(x, W1.T, degp)

    agg5 = _make_agg(5)
    s1p = agg5(g1.reshape(-1), src, dst).reshape(NC, 5, NP)
    x1t, g2 = _tc_layer1(s1p, dinv, g1, b1.reshape(5, 1))

    s2p = agg5(g2.reshape(-1), src, dst).reshape(NC, 5, NP)
    x2t, g3 = _tc_layer2(s2p, dinv, g2, W2.T, b2.reshape(6, 1))

    s3p = _make_agg(6)(g3.reshape(-1), src, dst).reshape(NC, 6, NP)
    x3t, outt = _tc_layer3(s3p, dinv, g3, W3.T, b3.reshape(7, 1),
                           Wl.T, bl.reshape(8, 1))

    return (outt[:, :N].T, [x1t[:, :N].T, x2t[:, :N].T, x3t[:, :N].T])
